# Initial kernel scaffold; baseline (speedup 1.0000x reference)
import jax
import jax.numpy as jnp
from jax.experimental import pallas as pl


def kernel(x, edge_index, edge_attr, batch, W1, b1, W2, b2, We, be, eps):
    raise NotImplementedError("write your pallas kernel here")


# speedup vs baseline: 3.7024x; 3.7024x over previous
import functools

import jax
import jax.numpy as jnp
from jax import lax
from jax.experimental import pallas as pl
from jax.experimental.pallas import tpu as pltpu
from jax.experimental.pallas import tpu_sc as plsc

NS = 16
NC = 2
LN = 16


def _sc_mesh():
    return plsc.VectorSubcoreMesh(
        core_axis_name="c", subcore_axis_name="s",
        num_cores=NC, num_subcores=NS)


def _make_inv_kernel(n_pad, nsj, njc, kb):
    npt = n_pad // NS
    nrc = npt // 128

    @functools.partial(
        pl.kernel,
        out_type=jax.ShapeDtypeStruct((n_pad,), jnp.float32),
        mesh=_sc_mesh(),
        scratch_types=[
            pltpu.VMEM((njc, kb), jnp.int32),
            pltpu.VMEM((kb,), jnp.float32),
            pltpu.VMEM((128,), jnp.float32),
            pltpu.VMEM_SHARED((n_pad,), jnp.float32),
        ],
    )
    def inv_kernel(dst_hbm, inv_hbm, dst_t, ones_t, cbuf, cnt_sh):
        c = lax.axis_index("c")
        s = lax.axis_index("s")
        for v in range(kb // LN):
            ones_t[pl.ds(v * LN, LN)] = jnp.ones((LN,), jnp.float32)
        for v in range(128 // LN):
            cbuf[pl.ds(v * LN, LN)] = jnp.zeros((LN,), jnp.float32)
        for k in range(nrc):
            pltpu.sync_copy(cbuf, cnt_sh.at[pl.ds(npt * s + 128 * k, 128)])
        plsc.subcore_barrier()

        def superchunk(jj, carry):
            pltpu.sync_copy(dst_hbm.at[s, jj], dst_t)

            def body(j, cc):
                pltpu.sync_copy(ones_t, cnt_sh.at[dst_t.at[j]], add=True)
                return cc
            lax.fori_loop(0, njc, body, 0)
            return carry
        lax.fori_loop(0, nsj, superchunk, 0)
        plsc.subcore_barrier()

        for k in range(nrc):
            base = npt * s + 128 * k
            pltpu.sync_copy(cnt_sh.at[pl.ds(base, 128)], cbuf)
            for v in range(128 // LN):
                sl = pl.ds(v * LN, LN)
                cbuf[sl] = 1.0 / jnp.maximum(cbuf[sl], 1.0)

            @pl.when(c == 0)
            def _():
                pltpu.sync_copy(cbuf, inv_hbm.at[pl.ds(base, 128)])

    return inv_kernel


def _make_agg_kernel(n_pad, dh, nj, kb):
    npt = n_pad // NS
    nrc = npt // 128
    nv = dh // LN
    njc = 25
    nsj = nj // njc

    @functools.partial(
        pl.kernel,
        out_type=jax.ShapeDtypeStruct((NC * n_pad, dh), jnp.float32),
        mesh=_sc_mesh(),
        scratch_types=[
            pltpu.VMEM((njc, kb), jnp.int32),
            pltpu.VMEM((njc, kb), jnp.int32),
            pltpu.VMEM((njc, kb), jnp.float32),
            pltpu.VMEM((kb, dh), jnp.float32),
            pltpu.VMEM((dh,), jnp.float32),
            pltpu.VMEM((dh,), jnp.float32),
            pltpu.VMEM((128,), jnp.float32),
            pltpu.VMEM((128, dh), jnp.float32),
            pltpu.VMEM_SHARED((n_pad, dh), jnp.float32),
            pltpu.SemaphoreType.DMA,
        ],
    )
    def agg_kernel(h_hbm, src_hbm, dst_hbm, ea_hbm, we_hbm, be_hbm, inv_hbm,
                   agg_hbm, src_t, dst_t, ea_t, rows, webuf, bebuf, invbuf,
                   rowbuf, agg_sh, gsem):
        c = lax.axis_index("c")
        s = lax.axis_index("s")
        pltpu.sync_copy(we_hbm.at[c], webuf)
        pltpu.sync_copy(be_hbm.at[c], bebuf)

        off = c * n_pad

        def zrow(r, carry):
            for v in range(nv):
                rowbuf[r, pl.ds(v * LN, LN)] = jnp.zeros((LN,), jnp.float32)
            return carry
        lax.fori_loop(0, 128, zrow, 0)
        for k in range(nrc):
            pltpu.sync_copy(rowbuf, agg_sh.at[pl.ds(npt * s + 128 * k, 128)])
        plsc.subcore_barrier()

        ws = [webuf[pl.ds(v * LN, LN)] for v in range(nv)]
        bs = [bebuf[pl.ds(v * LN, LN)] for v in range(nv)]

        def superchunk(jj, carry):
            pltpu.sync_copy(src_hbm.at[s, jj], src_t)
            pltpu.sync_copy(dst_hbm.at[s, jj], dst_t)
            pltpu.sync_copy(ea_hbm.at[s, jj], ea_t)

            def shift(j, cc):
                for v in range(kb // LN):
                    sl = pl.ds(v * LN, LN)
                    src_t[j, sl] = src_t[j, sl] + off
                return cc
            lax.fori_loop(0, njc, shift, 0)

            def chunk(j, cc):
                pltpu.async_copy(h_hbm.at[src_t.at[j]], rows, gsem).wait()

                def egroup(g, ccc):
                    eav = ea_t[j, pl.ds(g * LN, LN)]
                    for i in range(LN):
                        ea = eav[i]
                        e_row = g * LN + i
                        for v in range(nv):
                            sl = pl.ds(v * LN, LN)
                            rows[e_row, sl] = jnp.maximum(
                                rows[e_row, sl] + ea * ws[v] + bs[v], 0.0)
                    return ccc
                lax.fori_loop(0, kb // LN, egroup, 0)
                pltpu.sync_copy(rows, agg_sh.at[dst_t.at[j]], add=True)
                return cc
            lax.fori_loop(0, njc, chunk, 0)
            return carry
        lax.fori_loop(0, nsj, superchunk, 0)
        plsc.subcore_barrier()

        for k in range(nrc):
            base = npt * s + 128 * k
            pltpu.sync_copy(inv_hbm.at[pl.ds(base, 128)], invbuf)
            pltpu.sync_copy(agg_sh.at[pl.ds(base, 128)], rowbuf)

            def srow(g, carry):
                ivv = invbuf[pl.ds(g * LN, LN)]
                for i in range(LN):
                    iv = ivv[i]
                    for v in range(nv):
                        sl = pl.ds(v * LN, LN)
                        rowbuf[g * LN + i, sl] = rowbuf[g * LN + i, sl] * iv
                return carry
            lax.fori_loop(0, 128 // LN, srow, 0)
            pltpu.sync_copy(rowbuf, agg_hbm.at[pl.ds(off + base, 128)])

    return agg_kernel


def _mlp_body(h_ref, a_ref, w1_ref, b1_ref, w2_ref, b2_ref, eps_ref, o_ref):
    dh = h_ref.shape[2]
    eps_v = eps_ref[0]
    h = jnp.concatenate([h_ref[0], h_ref[1]], axis=1)
    a = jnp.concatenate([a_ref[0], a_ref[1]], axis=1)
    pre = (1.0 + eps_v) * h + a
    t = jnp.maximum(
        jnp.dot(pre, w1_ref[...], preferred_element_type=jnp.float32)
        + b1_ref[...], 0.0)
    o = jnp.dot(t, w2_ref[...], preferred_element_type=jnp.float32) + b2_ref[...]
    hn = jnp.maximum(o, 0.0)
    o_ref[0] = hn[:, :dh]
    o_ref[1] = hn[:, dh:]


def _mlp_call(h3, a3, w1, b1r, w2, b2r, epsl, n_pad, dh, blk):
    d = 2 * dh
    grid = n_pad // blk
    return pl.pallas_call(
        _mlp_body,
        grid=(grid,),
        in_specs=[
            pl.BlockSpec((2, blk, dh), lambda i: (0, i, 0)),
            pl.BlockSpec((2, blk, dh), lambda i: (0, i, 0)),
            pl.BlockSpec((d, d), lambda i: (0, 0)),
            pl.BlockSpec((1, d), lambda i: (0, 0)),
            pl.BlockSpec((d, d), lambda i: (0, 0)),
            pl.BlockSpec((1, d), lambda i: (0, 0)),
            pl.BlockSpec(memory_space=pltpu.SMEM),
        ],
        out_specs=pl.BlockSpec((2, blk, dh), lambda i: (0, i, 0)),
        out_shape=jax.ShapeDtypeStruct((2, n_pad, dh), jnp.float32),
    )(h3, a3, w1, b1r, w2, b2r, epsl)


def kernel(x, edge_index, edge_attr, batch, W1, b1, W2, b2, We, be, eps):
    n, d = x.shape
    e = edge_index.shape[1]
    nl = W1.shape[0]
    dh = d // 2
    n_pad = ((n + 128 * NS - 1) // (128 * NS)) * (128 * NS)
    ept = e // NS
    kb = 80
    nj = ept // kb
    blk = 512

    njc = 25
    nsj = nj // njc
    src_t = edge_index[0].reshape(NS, nsj, njc, kb)
    dst_t = edge_index[1].reshape(NS, nsj, njc, kb)
    ea_t = edge_attr.reshape(NS, nsj, njc, kb)

    inv = _make_inv_kernel(n_pad, nsj, njc, kb)(dst_t)
    agg_fn = _make_agg_kernel(n_pad, dh, nj, kb)

    xp = jnp.zeros((2, n_pad, dh), jnp.float32)
    xp = xp.at[0, :n].set(x[:, :dh]).at[1, :n].set(x[:, dh:])
    h2 = xp.reshape(2 * n_pad, dh)

    outs = []
    for l in range(nl):
        wrow = jnp.stack([We[l, 0, :dh], We[l, 0, dh:]])
        brow = jnp.stack([be[l, :dh], be[l, dh:]])
        agg = agg_fn(h2, src_t, dst_t, ea_t, wrow, brow, inv)
        h3 = _mlp_call(h2.reshape(2, n_pad, dh), agg.reshape(2, n_pad, dh),
                       W1[l], b1[l].reshape(1, d), W2[l], b2[l].reshape(1, d),
                       eps[l].reshape(1), n_pad, dh, blk)
        h2 = h3.reshape(2 * n_pad, dh)
        outs.append(h3)

    return jnp.concatenate(
        [jnp.concatenate([h3[0, :n], h3[1, :n]], axis=1) for h3 in outs],
        axis=1)


# speedup vs baseline: 4.5815x; 1.2374x over previous
import functools

import jax
import jax.numpy as jnp
from jax import lax
from jax.experimental import pallas as pl
from jax.experimental.pallas import tpu as pltpu
from jax.experimental.pallas import tpu_sc as plsc

NS = 16
NC = 2
LN = 16


def _sc_mesh():
    return plsc.VectorSubcoreMesh(
        core_axis_name="c", subcore_axis_name="s",
        num_cores=NC, num_subcores=NS)


def _make_inv_kernel(n_pad, nsj, njc, kb):
    npt = n_pad // NS
    nrc = npt // 128

    @functools.partial(
        pl.kernel,
        out_type=jax.ShapeDtypeStruct((n_pad,), jnp.float32),
        mesh=_sc_mesh(),
        scratch_types=[
            pltpu.VMEM((njc, kb), jnp.int32),
            pltpu.VMEM((kb,), jnp.float32),
            pltpu.VMEM((128,), jnp.float32),
            pltpu.VMEM_SHARED((n_pad,), jnp.float32),
        ],
    )
    def inv_kernel(dst_hbm, inv_hbm, dst_t, ones_t, cbuf, cnt_sh):
        c = lax.axis_index("c")
        s = lax.axis_index("s")
        for v in range(kb // LN):
            ones_t[pl.ds(v * LN, LN)] = jnp.ones((LN,), jnp.float32)
        for v in range(128 // LN):
            cbuf[pl.ds(v * LN, LN)] = jnp.zeros((LN,), jnp.float32)
        for k in range(nrc):
            pltpu.sync_copy(cbuf, cnt_sh.at[pl.ds(npt * s + 128 * k, 128)])
        plsc.subcore_barrier()

        def superchunk(jj, carry):
            pltpu.sync_copy(dst_hbm.at[s, jj], dst_t)

            def body(j, cc):
                pltpu.sync_copy(ones_t, cnt_sh.at[dst_t.at[j]], add=True)
                return cc
            lax.fori_loop(0, njc, body, 0)
            return carry
        lax.fori_loop(0, nsj, superchunk, 0)
        plsc.subcore_barrier()

        for k in range(nrc):
            base = npt * s + 128 * k
            pltpu.sync_copy(cnt_sh.at[pl.ds(base, 128)], cbuf)
            for v in range(128 // LN):
                sl = pl.ds(v * LN, LN)
                cbuf[sl] = 1.0 / jnp.maximum(cbuf[sl], 1.0)

            @pl.when(c == 0)
            def _():
                pltpu.sync_copy(cbuf, inv_hbm.at[pl.ds(base, 128)])

    return inv_kernel


def _make_agg_kernel(n_pad, dh, nsj, njc, kb):
    npt = n_pad // NS
    wrc = 64
    nrc = npt // wrc
    nv = dh // LN
    nbuf = 3
    assert njc % nbuf == 0

    @functools.partial(
        pl.kernel,
        out_type=jax.ShapeDtypeStruct((NC * n_pad, dh), jnp.float32),
        mesh=_sc_mesh(),
        scratch_types=[
            pltpu.VMEM((njc, kb), jnp.int32),
            pltpu.VMEM((njc, kb), jnp.int32),
            pltpu.VMEM((njc, kb), jnp.float32),
            [pltpu.VMEM((kb, dh), jnp.float32) for _ in range(nbuf)],
            pltpu.VMEM((dh,), jnp.float32),
            pltpu.VMEM((dh,), jnp.float32),
            pltpu.VMEM((wrc,), jnp.float32),
            pltpu.VMEM((wrc, dh), jnp.float32),
            pltpu.VMEM_SHARED((n_pad, dh), jnp.float32),
            [pltpu.SemaphoreType.DMA for _ in range(nbuf)],
            [pltpu.SemaphoreType.DMA for _ in range(nbuf)],
        ],
    )
    def agg_kernel(h_hbm, src_hbm, dst_hbm, ea_hbm, we_hbm, be_hbm, inv_hbm,
                   agg_hbm, src_t, dst_t, ea_t, rows, webuf, bebuf, invbuf,
                   rowbuf, agg_sh, gsems, ssems):
        c = lax.axis_index("c")
        s = lax.axis_index("s")
        pltpu.sync_copy(we_hbm.at[c], webuf)
        pltpu.sync_copy(be_hbm.at[c], bebuf)

        off = c * n_pad

        def zrow(r, carry):
            for v in range(nv):
                rowbuf[r, pl.ds(v * LN, LN)] = jnp.zeros((LN,), jnp.float32)
            return carry
        lax.fori_loop(0, wrc, zrow, 0)
        for k in range(nrc):
            pltpu.sync_copy(rowbuf, agg_sh.at[pl.ds(npt * s + wrc * k, wrc)])
        plsc.subcore_barrier()

        ws = [webuf[pl.ds(v * LN, LN)] for v in range(nv)]
        bs = [bebuf[pl.ds(v * LN, LN)] for v in range(nv)]

        def gather(j, b):
            pltpu.async_copy(h_hbm.at[src_t.at[j]], rows[b], gsems[b])

        def wait_gather(b):
            pltpu.make_async_copy(h_hbm.at[src_t.at[0]], rows[b],
                                  gsems[b]).wait()

        def scatter(j, b):
            pltpu.async_copy(rows[b], agg_sh.at[dst_t.at[j]], ssems[b],
                             add=True)

        def wait_scatter(b):
            pltpu.make_async_copy(rows[b], agg_sh.at[dst_t.at[0]],
                                  ssems[b]).wait()

        def compute(j, b):
            def egroup(g, ccc):
                eav = ea_t[j, pl.ds(g * LN, LN)]
                for i in range(LN):
                    ea = eav[i]
                    e_row = g * LN + i
                    for v in range(nv):
                        sl = pl.ds(v * LN, LN)
                        rows[b][e_row, sl] = jnp.maximum(
                            rows[b][e_row, sl] + ea * ws[v] + bs[v], 0.0)
                return ccc
            lax.fori_loop(0, kb // LN, egroup, 0)

        def superchunk(jj, carry):
            pltpu.sync_copy(src_hbm.at[s, jj], src_t)
            pltpu.sync_copy(dst_hbm.at[s, jj], dst_t)
            pltpu.sync_copy(ea_hbm.at[s, jj], ea_t)

            def shift(j, cc):
                for v in range(kb // LN):
                    sl = pl.ds(v * LN, LN)
                    src_t[j, sl] = src_t[j, sl] + off
                return cc
            lax.fori_loop(0, njc, shift, 0)

            gather(0, 0)
            gather(1, 1)

            def triple(q, cc):
                for b in range(nbuf):
                    j = q * nbuf + b
                    wait_gather(b)
                    compute(j, b)
                    scatter(j, b)
                    bf = (b + 2) % nbuf
                    jf = j + 2

                    @pl.when((jf < njc) & (j >= 1))
                    def _():
                        wait_scatter(bf)

                    @pl.when(jf < njc)
                    def _():
                        gather(jf, bf)
                return cc
            lax.fori_loop(0, njc // nbuf, triple, 0)
            for b in range(nbuf):
                wait_scatter(b)
            return carry
        lax.fori_loop(0, nsj, superchunk, 0)
        plsc.subcore_barrier()

        for k in range(nrc):
            base = npt * s + wrc * k
            pltpu.sync_copy(inv_hbm.at[pl.ds(base, wrc)], invbuf)
            pltpu.sync_copy(agg_sh.at[pl.ds(base, wrc)], rowbuf)

            def srow(g, carry):
                ivv = invbuf[pl.ds(g * LN, LN)]
                for i in range(LN):
                    iv = ivv[i]
                    for v in range(nv):
                        sl = pl.ds(v * LN, LN)
                        rowbuf[g * LN + i, sl] = rowbuf[g * LN + i, sl] * iv
                return carry
            lax.fori_loop(0, wrc // LN, srow, 0)
            pltpu.sync_copy(rowbuf, agg_hbm.at[pl.ds(off + base, wrc)])

    return agg_kernel


def _mlp_body(h_ref, a_ref, w1_ref, b1_ref, w2_ref, b2_ref, eps_ref, o_ref):
    dh = h_ref.shape[2]
    eps_v = eps_ref[0]
    h = jnp.concatenate([h_ref[0], h_ref[1]], axis=1)
    a = jnp.concatenate([a_ref[0], a_ref[1]], axis=1)
    pre = (1.0 + eps_v) * h + a
    t = jnp.maximum(
        jnp.dot(pre, w1_ref[...], preferred_element_type=jnp.float32)
        + b1_ref[...], 0.0)
    o = jnp.dot(t, w2_ref[...], preferred_element_type=jnp.float32) + b2_ref[...]
    hn = jnp.maximum(o, 0.0)
    o_ref[0] = hn[:, :dh]
    o_ref[1] = hn[:, dh:]


def _mlp_call(h3, a3, w1, b1r, w2, b2r, epsl, n_pad, dh, blk):
    d = 2 * dh
    grid = n_pad // blk
    return pl.pallas_call(
        _mlp_body,
        grid=(grid,),
        in_specs=[
            pl.BlockSpec((2, blk, dh), lambda i: (0, i, 0)),
            pl.BlockSpec((2, blk, dh), lambda i: (0, i, 0)),
            pl.BlockSpec((d, d), lambda i: (0, 0)),
            pl.BlockSpec((1, d), lambda i: (0, 0)),
            pl.BlockSpec((d, d), lambda i: (0, 0)),
            pl.BlockSpec((1, d), lambda i: (0, 0)),
            pl.BlockSpec(memory_space=pltpu.SMEM),
        ],
        out_specs=pl.BlockSpec((2, blk, dh), lambda i: (0, i, 0)),
        out_shape=jax.ShapeDtypeStruct((2, n_pad, dh), jnp.float32),
    )(h3, a3, w1, b1r, w2, b2r, epsl)


def kernel(x, edge_index, edge_attr, batch, W1, b1, W2, b2, We, be, eps):
    n, d = x.shape
    e = edge_index.shape[1]
    nl = W1.shape[0]
    dh = d // 2
    n_pad = ((n + 128 * NS - 1) // (128 * NS)) * (128 * NS)
    ept = e // NS
    kb = 80
    njc = 21
    blk = 512
    nsj = -(-ept // (njc * kb))
    pad = nsj * njc * kb - ept

    src2 = edge_index[0].reshape(NS, ept)
    dst2 = edge_index[1].reshape(NS, ept)
    ea2 = edge_attr.reshape(NS, ept)
    if pad:
        src2 = jnp.concatenate(
            [src2, jnp.zeros((NS, pad), jnp.int32)], axis=1)
        dst2 = jnp.concatenate(
            [dst2, jnp.full((NS, pad), n_pad - 1, jnp.int32)], axis=1)
        ea2 = jnp.concatenate(
            [ea2, jnp.zeros((NS, pad), jnp.float32)], axis=1)
    src_t = src2.reshape(NS, nsj, njc, kb)
    dst_t = dst2.reshape(NS, nsj, njc, kb)
    ea_t = ea2.reshape(NS, nsj, njc, kb)

    inv = _make_inv_kernel(n_pad, nsj, njc, kb)(dst_t)
    agg_fn = _make_agg_kernel(n_pad, dh, nsj, njc, kb)

    xp = jnp.zeros((2, n_pad, dh), jnp.float32)
    xp = xp.at[0, :n].set(x[:, :dh]).at[1, :n].set(x[:, dh:])
    h2 = xp.reshape(2 * n_pad, dh)

    outs = []
    for l in range(nl):
        wrow = jnp.stack([We[l, 0, :dh], We[l, 0, dh:]])
        brow = jnp.stack([be[l, :dh], be[l, dh:]])
        agg = agg_fn(h2, src_t, dst_t, ea_t, wrow, brow, inv)
        h3 = _mlp_call(h2.reshape(2, n_pad, dh), agg.reshape(2, n_pad, dh),
                       W1[l], b1[l].reshape(1, d), W2[l], b2[l].reshape(1, d),
                       eps[l].reshape(1), n_pad, dh, blk)
        h2 = h3.reshape(2 * n_pad, dh)
        outs.append(h3)

    return jnp.concatenate(
        [jnp.concatenate([h3[0, :n], h3[1, :n]], axis=1) for h3 in outs],
        axis=1)
